# baseline (speedup 1.0000x reference)
import functools

import jax
import jax.numpy as jnp
from jax import lax
from jax.experimental import pallas as pl
from jax.experimental.pallas import tpu as pltpu
from jax.experimental.pallas import tpu_sc as plsc

N_NODES = 10000
N_EDGES = 320000
D_FEAT = 128
D_EDGE = 16
HIDDEN = 64

NC = 2
NS = 16
NW = NC * NS
NPAD = 10240
RPT = NPAD // NS
EPW = N_EDGES // NW
CH = 400
NCHUNK = EPW // CH
ZR = 160

BLK = 1000
NBLK = N_NODES // BLK


def _sc_mesh():
    return plsc.VectorSubcoreMesh(core_axis_name="c", subcore_axis_name="s")


_SC_PARAMS = pltpu.CompilerParams(use_tc_tiling_on_sc=False)


def _fill_rows(buf, rows, cols, value):
    v = jnp.full((16,), value, jnp.float32)

    def body(i, carry):
        for k in range(cols // 16):
            buf[i, pl.ds(k * 16, 16)] = v
        return carry

    lax.fori_loop(0, rows, body, 0)


CH1 = 2000
NCHUNK1 = EPW // CH1


def _edge_agg_sc(row, edge_attr):
    out_type = [
        jax.ShapeDtypeStruct((NC, NPAD, D_EDGE), jnp.float32),
        jax.ShapeDtypeStruct((NC, NPAD, 16), jnp.float32),
    ]
    scratch = [
        pltpu.VMEM((EPW,), jnp.int32),
        pltpu.VMEM((CH1, D_EDGE), jnp.float32),
        pltpu.VMEM((CH1, D_EDGE), jnp.float32),
        pltpu.VMEM((CH1, 16), jnp.float32),
        pltpu.VMEM((ZR, 16), jnp.float32),
        pltpu.VMEM_SHARED((NPAD, D_EDGE), jnp.float32),
        pltpu.VMEM_SHARED((NPAD, 16), jnp.float32),
        pltpu.SemaphoreType.DMA,
        pltpu.SemaphoreType.DMA,
    ]

    @functools.partial(pl.kernel, out_type=out_type, mesh=_sc_mesh(),
                       scratch_types=scratch, compiler_params=_SC_PARAMS)
    def k(row_h, ea_h, out_e, out_c, rix, e0, e1, obuf, zbuf,
          acc_e, acc_c, sl0, sl1):
        cid = lax.axis_index("c")
        sid = lax.axis_index("s")
        w = cid * NS + sid
        eb, sl = (e0, e1), (sl0, sl1)
        pltpu.sync_copy(row_h.at[pl.ds(w * EPW, EPW)], rix)
        _fill_rows(zbuf, ZR, 16, 0.0)
        _fill_rows(obuf, CH1, 16, 1.0)
        r0 = sid * RPT
        for q in range(RPT // ZR):
            pltpu.sync_copy(zbuf, acc_e.at[pl.ds(r0 + q * ZR, ZR)])
            pltpu.sync_copy(zbuf, acc_c.at[pl.ds(r0 + q * ZR, ZR)])
        plsc.subcore_barrier()
        base0 = w * EPW
        dl = [None] * NCHUNK1
        dl[0] = pltpu.async_copy(ea_h.at[pl.ds(base0, CH1)], eb[0], sl[0])
        for j in range(NCHUNK1):
            b = j % 2
            dl[j].wait()
            if j + 1 < NCHUNK1:
                dl[j + 1] = pltpu.async_copy(
                    ea_h.at[pl.ds(base0 + (j + 1) * CH1, CH1)], eb[1 - b],
                    sl[1 - b])
            ridx = rix.at[pl.ds(j * CH1, CH1)]
            pltpu.sync_copy(eb[b], acc_e.at[ridx], add=True)
            pltpu.sync_copy(obuf, acc_c.at[ridx], add=True)
        plsc.subcore_barrier()
        pltpu.sync_copy(acc_e.at[pl.ds(r0, RPT)], out_e.at[cid, pl.ds(r0, RPT)])
        pltpu.sync_copy(acc_c.at[pl.ds(r0, RPT)], out_c.at[cid, pl.ds(r0, RPT)])

    return k(row, edge_attr)


def _seg_sum_sc(row, col, table, d=HIDDEN, ch=CH):
    nchunk = EPW // ch
    out_type = jax.ShapeDtypeStruct((NC, NPAD, d), jnp.float32)
    scratch = [
        pltpu.VMEM((EPW,), jnp.int32),
        pltpu.VMEM((EPW,), jnp.int32),
        pltpu.VMEM((ch, d), jnp.float32),
        pltpu.VMEM((ch, d), jnp.float32),
        pltpu.VMEM((ZR, d), jnp.float32),
        pltpu.VMEM_SHARED((NPAD, d), jnp.float32),
        pltpu.SemaphoreType.DMA,
        pltpu.SemaphoreType.DMA,
        pltpu.SemaphoreType.DMA,
        pltpu.SemaphoreType.DMA,
    ]

    @functools.partial(pl.kernel, out_type=out_type, mesh=_sc_mesh(),
                       scratch_types=scratch, compiler_params=_SC_PARAMS)
    def k(row_h, col_h, tab, out, rix, cix, g0, g1, zbuf, acc, sg0, sg1,
          ss0, ss1):
        cid = lax.axis_index("c")
        sid = lax.axis_index("s")
        w = cid * NS + sid
        r0 = sid * RPT
        gb, sg, ss = (g0, g1), (sg0, sg1), (ss0, ss1)
        base0 = w * EPW
        pltpu.sync_copy(row_h.at[pl.ds(base0, EPW)], rix)
        pltpu.sync_copy(col_h.at[pl.ds(base0, EPW)], cix)
        _fill_rows(zbuf, ZR, d, 0.0)
        for q in range(RPT // ZR):
            pltpu.sync_copy(zbuf, acc.at[pl.ds(r0 + q * ZR, ZR)])
        plsc.subcore_barrier()
        dg = [None] * nchunk
        ds = [None] * nchunk
        dg[0] = pltpu.async_copy(tab.at[rix.at[pl.ds(0, ch)]], gb[0], sg[0])
        for j in range(nchunk):
            b = j % 2
            dg[j].wait()
            if j + 1 < nchunk:
                if j >= 1:
                    ds[j - 1].wait()
                dg[j + 1] = pltpu.async_copy(
                    tab.at[rix.at[pl.ds((j + 1) * ch, ch)]],
                    gb[1 - b], sg[1 - b])
            ds[j] = pltpu.async_copy(gb[b], acc.at[cix.at[pl.ds(j * ch, ch)]],
                                     ss[b], add=True)
        ds[nchunk - 1].wait()
        if nchunk > 1:
            ds[nchunk - 2].wait()
        plsc.subcore_barrier()
        pltpu.sync_copy(acc.at[pl.ds(r0, RPT)], out.at[cid, pl.ds(r0, RPT)])

    return k(row, col, table)


def _ux_tc(x, w1a):
    def body(x_ref, w1a_ref, u_ref):
        u_ref[...] = jnp.dot(x_ref[...], w1a_ref[...],
                             preferred_element_type=jnp.float32)

    full = lambda shape: pl.BlockSpec(shape, lambda i: tuple(0 for _ in shape))
    return pl.pallas_call(
        body,
        grid=(NBLK,),
        in_specs=[
            pl.BlockSpec((BLK, D_FEAT), lambda i: (i, 0)),
            full((D_FEAT, HIDDEN)),
        ],
        out_specs=pl.BlockSpec((BLK, HIDDEN), lambda i: (i, 0)),
        out_shape=jax.ShapeDtypeStruct((N_NODES, HIDDEN), jnp.float32),
    )(x, w1a)


def _ue_tc(pe, pc, w1b):
    def body(pe_ref, pc_ref, w1b_ref, ue_ref):
        s = pe_ref[0] + pe_ref[1]
        c = pc_ref[0] + pc_ref[1]
        em = s / jnp.maximum(c[:, 0:1], 1.0)
        ue_ref[...] = jnp.dot(em, w1b_ref[...],
                              preferred_element_type=jnp.float32)

    full = lambda shape: pl.BlockSpec(shape, lambda i: tuple(0 for _ in shape))
    return pl.pallas_call(
        body,
        grid=(NBLK,),
        in_specs=[
            pl.BlockSpec((NC, BLK, D_EDGE), lambda i: (0, i, 0)),
            pl.BlockSpec((NC, BLK, 16), lambda i: (0, i, 0)),
            full((D_EDGE, HIDDEN)),
        ],
        out_specs=pl.BlockSpec((BLK, HIDDEN), lambda i: (i, 0)),
        out_shape=jax.ShapeDtypeStruct((N_NODES, HIDDEN), jnp.float32),
    )(pe, pc, w1b)


def _layer1_tc(ux, ue, sx, se, b1, w2, b2, w1n):
    def body(ux_ref, ue_ref, sx_ref, se_ref, b1_ref, w2_ref,
             b2_ref, w1n_ref, un_ref):
        z = (ux_ref[...] + ue_ref[...] + sx_ref[0] + sx_ref[1]
             + se_ref[0] + se_ref[1] + b1_ref[...])
        a = jnp.maximum(z, 0.0)
        t = jnp.dot(a, w2_ref[...], preferred_element_type=jnp.float32) + b2_ref[...]
        h = jnp.maximum(t, 0.0)
        un_ref[...] = jnp.dot(h, w1n_ref[...], preferred_element_type=jnp.float32)

    full = lambda shape: pl.BlockSpec(shape, lambda i: tuple(0 for _ in shape))
    return pl.pallas_call(
        body,
        grid=(NBLK,),
        in_specs=[
            pl.BlockSpec((BLK, HIDDEN), lambda i: (i, 0)),
            pl.BlockSpec((BLK, HIDDEN), lambda i: (i, 0)),
            pl.BlockSpec((NC, BLK, HIDDEN), lambda i: (0, i, 0)),
            pl.BlockSpec((NC, BLK, HIDDEN), lambda i: (0, i, 0)),
            full((1, HIDDEN)),
            full((HIDDEN, HIDDEN)),
            full((1, HIDDEN)),
            full((HIDDEN, HIDDEN)),
        ],
        out_specs=pl.BlockSpec((BLK, HIDDEN), lambda i: (i, 0)),
        out_shape=jax.ShapeDtypeStruct((N_NODES, HIDDEN), jnp.float32),
    )(ux, ue, sx, se, b1, w2, b2, w1n)


def _layer_tc(u, s, b1, w2, b2, w1n):
    def body(u_ref, s_ref, b1_ref, w2_ref, b2_ref, w1n_ref, un_ref):
        z = u_ref[...] + s_ref[0] + s_ref[1] + b1_ref[...]
        a = jnp.maximum(z, 0.0)
        t = jnp.dot(a, w2_ref[...], preferred_element_type=jnp.float32) + b2_ref[...]
        h = jnp.maximum(t, 0.0)
        un_ref[...] = jnp.dot(h, w1n_ref[...], preferred_element_type=jnp.float32)

    full = lambda shape: pl.BlockSpec(shape, lambda i: tuple(0 for _ in shape))
    return pl.pallas_call(
        body,
        grid=(NBLK,),
        in_specs=[
            pl.BlockSpec((BLK, HIDDEN), lambda i: (i, 0)),
            pl.BlockSpec((NC, BLK, HIDDEN), lambda i: (0, i, 0)),
            full((1, HIDDEN)),
            full((HIDDEN, HIDDEN)),
            full((1, HIDDEN)),
            full((HIDDEN, HIDDEN)),
        ],
        out_specs=pl.BlockSpec((BLK, HIDDEN), lambda i: (i, 0)),
        out_shape=jax.ShapeDtypeStruct((N_NODES, HIDDEN), jnp.float32),
    )(u, s, b1, w2, b2, w1n)


def _pool_clf_tc(h, wc1, bc1, wc2, bc2):
    def body(h_ref, wc1_ref, bc1_ref, wc2_ref, bc2_ref, o_ref, acc):
        i = pl.program_id(0)

        @pl.when(i == 0)
        def _():
            acc[...] = jnp.zeros_like(acc)

        acc[...] += jnp.sum(h_ref[...], axis=0, keepdims=True)

        @pl.when(i == NBLK - 1)
        def _():
            pooled = acc[...] * (1.0 / N_NODES)
            l1 = jnp.maximum(
                jnp.dot(pooled, wc1_ref[...], preferred_element_type=jnp.float32)
                + bc1_ref[...], 0.0)
            o_ref[...] = (jnp.dot(l1, wc2_ref[...],
                                  preferred_element_type=jnp.float32)
                          + bc2_ref[...])

    full = lambda shape: pl.BlockSpec(shape, lambda i: tuple(0 for _ in shape))
    return pl.pallas_call(
        body,
        grid=(NBLK,),
        in_specs=[
            pl.BlockSpec((BLK, HIDDEN), lambda i: (i, 0)),
            full((HIDDEN, 128)),
            full((1, 128)),
            full((128, 128)),
            full((1, 128)),
        ],
        out_specs=pl.BlockSpec((1, 128), lambda i: (0, 0)),
        out_shape=jax.ShapeDtypeStruct((1, 128), jnp.float32),
        scratch_shapes=[pltpu.VMEM((1, HIDDEN), jnp.float32)],
    )(h, wc1, bc1, wc2, bc2)


def _fold_bn(p):
    s = p['gamma'] * jax.lax.rsqrt(p['rv'] + 1e-5)
    w2 = p['W2'] * s[None, :]
    b2 = p['b2'] * s + p['beta'] - p['rm'] * s
    return w2, b2


def kernel(x, edge_attr, params, edge_index, batch):
    row = edge_index[0]
    col = edge_index[1]
    layers = params['layers']
    p0 = layers[0]
    w1a, w1b = p0['W1'][:D_FEAT], p0['W1'][D_FEAT:]
    w2b2 = [_fold_bn(p) for p in layers]
    eye = jnp.eye(HIDDEN, dtype=jnp.float32)

    ux = _ux_tc(x, w1a)
    sx = _seg_sum_sc(row, col, ux)
    pe, pc = _edge_agg_sc(row, edge_attr)
    ue = _ue_tc(pe, pc, w1b)
    se = _seg_sum_sc(row, col, ue)
    u2 = _layer1_tc(ux, ue, sx, se, p0['b1'][None, :],
                    w2b2[0][0], w2b2[0][1][None, :], layers[1]['W1'])

    s2 = _seg_sum_sc(row, col, u2)
    u3 = _layer_tc(u2, s2, layers[1]['b1'][None, :],
                   w2b2[1][0], w2b2[1][1][None, :], layers[2]['W1'])
    s3 = _seg_sum_sc(row, col, u3)
    h3 = _layer_tc(u3, s3, layers[2]['b1'][None, :],
                   w2b2[2][0], w2b2[2][1][None, :], eye)

    c = params['clf']
    wc1 = jnp.zeros((HIDDEN, 128), jnp.float32).at[:, :HIDDEN // 2].set(c['W1'])
    bc1 = jnp.zeros((1, 128), jnp.float32).at[0, :HIDDEN // 2].set(c['b1'])
    wc2 = jnp.zeros((128, 128), jnp.float32).at[:HIDDEN // 2, :2].set(c['W2'])
    bc2 = jnp.zeros((1, 128), jnp.float32).at[0, :2].set(c['b2'])
    out = _pool_clf_tc(h3, wc1, bc1, wc2, bc2)
    return out[:, :2]


# speedup vs baseline: 1.0925x; 1.0925x over previous
import functools

import jax
import jax.numpy as jnp
from jax import lax
from jax.experimental import pallas as pl
from jax.experimental.pallas import tpu as pltpu
from jax.experimental.pallas import tpu_sc as plsc

N_NODES = 10000
N_EDGES = 320000
D_FEAT = 128
D_EDGE = 16
HIDDEN = 64

NC = 2
NS = 16
NW = NC * NS
NPAD = 10240
RPT = NPAD // NS
EPW = N_EDGES // NW
CH = 400
NCHUNK = EPW // CH
ZR = 160

BLK = 1000
NBLK = N_NODES // BLK


def _sc_mesh():
    return plsc.VectorSubcoreMesh(core_axis_name="c", subcore_axis_name="s")


_SC_PARAMS = pltpu.CompilerParams(use_tc_tiling_on_sc=False)


def _fill_rows(buf, rows, cols, value):
    v = jnp.full((16,), value, jnp.float32)

    def body(i, carry):
        for k in range(cols // 16):
            buf[i, pl.ds(k * 16, 16)] = v
        return carry

    lax.fori_loop(0, rows, body, 0)


CH1 = 2000
NCHUNK1 = EPW // CH1


def _edge_agg_sc(row, edge_attr):
    out_type = [
        jax.ShapeDtypeStruct((NC, NPAD, D_EDGE), jnp.float32),
        jax.ShapeDtypeStruct((NC, NPAD, 16), jnp.float32),
    ]
    scratch = [
        pltpu.VMEM((EPW,), jnp.int32),
        pltpu.VMEM((CH1, D_EDGE), jnp.float32),
        pltpu.VMEM((CH1, D_EDGE), jnp.float32),
        pltpu.VMEM((CH1, 16), jnp.float32),
        pltpu.VMEM((ZR, 16), jnp.float32),
        pltpu.VMEM_SHARED((NPAD, D_EDGE), jnp.float32),
        pltpu.VMEM_SHARED((NPAD, 16), jnp.float32),
        pltpu.SemaphoreType.DMA,
        pltpu.SemaphoreType.DMA,
    ]

    @functools.partial(pl.kernel, out_type=out_type, mesh=_sc_mesh(),
                       scratch_types=scratch, compiler_params=_SC_PARAMS)
    def k(row_h, ea_h, out_e, out_c, rix, e0, e1, obuf, zbuf,
          acc_e, acc_c, sl0, sl1):
        cid = lax.axis_index("c")
        sid = lax.axis_index("s")
        w = cid * NS + sid
        eb, sl = (e0, e1), (sl0, sl1)
        pltpu.sync_copy(row_h.at[pl.ds(w * EPW, EPW)], rix)
        _fill_rows(zbuf, ZR, 16, 0.0)
        _fill_rows(obuf, CH1, 16, 1.0)
        r0 = sid * RPT
        for q in range(RPT // ZR):
            pltpu.sync_copy(zbuf, acc_e.at[pl.ds(r0 + q * ZR, ZR)])
            pltpu.sync_copy(zbuf, acc_c.at[pl.ds(r0 + q * ZR, ZR)])
        plsc.subcore_barrier()
        base0 = w * EPW
        dl = [None] * NCHUNK1
        dl[0] = pltpu.async_copy(ea_h.at[pl.ds(base0, CH1)], eb[0], sl[0])
        for j in range(NCHUNK1):
            b = j % 2
            dl[j].wait()
            if j + 1 < NCHUNK1:
                dl[j + 1] = pltpu.async_copy(
                    ea_h.at[pl.ds(base0 + (j + 1) * CH1, CH1)], eb[1 - b],
                    sl[1 - b])
            ridx = rix.at[pl.ds(j * CH1, CH1)]
            pltpu.sync_copy(eb[b], acc_e.at[ridx], add=True)
            pltpu.sync_copy(obuf, acc_c.at[ridx], add=True)
        plsc.subcore_barrier()
        pltpu.sync_copy(acc_e.at[pl.ds(r0, RPT)], out_e.at[cid, pl.ds(r0, RPT)])
        pltpu.sync_copy(acc_c.at[pl.ds(r0, RPT)], out_c.at[cid, pl.ds(r0, RPT)])

    return k(row, edge_attr)


def _seg_sum_sc(row, col, table):
    d = HIDDEN
    out_type = jax.ShapeDtypeStruct((NC, NPAD, d), jnp.float32)
    scratch = [
        pltpu.VMEM((EPW,), jnp.int32),
        pltpu.VMEM((EPW,), jnp.int32),
        pltpu.VMEM((CH, d), jnp.float32),
        pltpu.VMEM((CH, d), jnp.float32),
        pltpu.VMEM((ZR, d), jnp.float32),
        pltpu.VMEM_SHARED((NPAD, d), jnp.float32),
        pltpu.SemaphoreType.DMA,
        pltpu.SemaphoreType.DMA,
        pltpu.SemaphoreType.DMA,
        pltpu.SemaphoreType.DMA,
    ]

    @functools.partial(pl.kernel, out_type=out_type, mesh=_sc_mesh(),
                       scratch_types=scratch, compiler_params=_SC_PARAMS)
    def k(row_h, col_h, tab, out, rix, cix, g0, g1, zbuf, acc, sg0, sg1,
          ss0, ss1):
        cid = lax.axis_index("c")
        sid = lax.axis_index("s")
        w = cid * NS + sid
        r0 = sid * RPT
        gb, sg, ss = (g0, g1), (sg0, sg1), (ss0, ss1)
        base0 = w * EPW
        pltpu.sync_copy(row_h.at[pl.ds(base0, EPW)], rix)
        pltpu.sync_copy(col_h.at[pl.ds(base0, EPW)], cix)
        _fill_rows(zbuf, ZR, d, 0.0)
        for q in range(RPT // ZR):
            pltpu.sync_copy(zbuf, acc.at[pl.ds(r0 + q * ZR, ZR)])
        plsc.subcore_barrier()
        dg = [None] * NCHUNK
        ds = [None] * NCHUNK
        dg[0] = pltpu.async_copy(tab.at[rix.at[pl.ds(0, CH)]], gb[0], sg[0])
        for j in range(NCHUNK):
            b = j % 2
            dg[j].wait()
            if j + 1 < NCHUNK:
                if j >= 1:
                    ds[j - 1].wait()
                dg[j + 1] = pltpu.async_copy(
                    tab.at[rix.at[pl.ds((j + 1) * CH, CH)]],
                    gb[1 - b], sg[1 - b])
            ds[j] = pltpu.async_copy(gb[b], acc.at[cix.at[pl.ds(j * CH, CH)]],
                                     ss[b], add=True)
        ds[NCHUNK - 1].wait()
        if NCHUNK > 1:
            ds[NCHUNK - 2].wait()
        plsc.subcore_barrier()
        pltpu.sync_copy(acc.at[pl.ds(r0, RPT)], out.at[cid, pl.ds(r0, RPT)])

    return k(row, col, table)


def _u1_tc(xp, pe, pc, w1a, w1b):
    def body(x_ref, pe_ref, pc_ref, w1a_ref, w1b_ref, u_ref):
        s = pe_ref[0] + pe_ref[1]
        c = pc_ref[0] + pc_ref[1]
        em = s / jnp.maximum(c[:, 0:1], 1.0)
        u_ref[...] = (jnp.dot(x_ref[...], w1a_ref[...],
                              preferred_element_type=jnp.float32)
                      + jnp.dot(em, w1b_ref[...],
                                preferred_element_type=jnp.float32))

    full = lambda shape: pl.BlockSpec(shape, lambda i: tuple(0 for _ in shape))
    return pl.pallas_call(
        body,
        grid=(NBLK,),
        in_specs=[
            pl.BlockSpec((BLK, D_FEAT), lambda i: (i, 0)),
            pl.BlockSpec((NC, BLK, D_EDGE), lambda i: (0, i, 0)),
            pl.BlockSpec((NC, BLK, 16), lambda i: (0, i, 0)),
            full((D_FEAT, HIDDEN)),
            full((D_EDGE, HIDDEN)),
        ],
        out_specs=pl.BlockSpec((BLK, HIDDEN), lambda i: (i, 0)),
        out_shape=jax.ShapeDtypeStruct((N_NODES, HIDDEN), jnp.float32),
    )(xp, pe, pc, w1a, w1b)


def _layer_tc(u, s, b1, w2, b2, w1n):
    def body(u_ref, s_ref, b1_ref, w2_ref, b2_ref, w1n_ref, un_ref):
        z = u_ref[...] + s_ref[0] + s_ref[1] + b1_ref[...]
        a = jnp.maximum(z, 0.0)
        t = jnp.dot(a, w2_ref[...], preferred_element_type=jnp.float32) + b2_ref[...]
        h = jnp.maximum(t, 0.0)
        un_ref[...] = jnp.dot(h, w1n_ref[...], preferred_element_type=jnp.float32)

    full = lambda shape: pl.BlockSpec(shape, lambda i: tuple(0 for _ in shape))
    return pl.pallas_call(
        body,
        grid=(NBLK,),
        in_specs=[
            pl.BlockSpec((BLK, HIDDEN), lambda i: (i, 0)),
            pl.BlockSpec((NC, BLK, HIDDEN), lambda i: (0, i, 0)),
            full((1, HIDDEN)),
            full((HIDDEN, HIDDEN)),
            full((1, HIDDEN)),
            full((HIDDEN, HIDDEN)),
        ],
        out_specs=pl.BlockSpec((BLK, HIDDEN), lambda i: (i, 0)),
        out_shape=jax.ShapeDtypeStruct((N_NODES, HIDDEN), jnp.float32),
    )(u, s, b1, w2, b2, w1n)


def _pool_clf_tc(h, wc1, bc1, wc2, bc2):
    def body(h_ref, wc1_ref, bc1_ref, wc2_ref, bc2_ref, o_ref, acc):
        i = pl.program_id(0)

        @pl.when(i == 0)
        def _():
            acc[...] = jnp.zeros_like(acc)

        acc[...] += jnp.sum(h_ref[...], axis=0, keepdims=True)

        @pl.when(i == NBLK - 1)
        def _():
            pooled = acc[...] * (1.0 / N_NODES)
            l1 = jnp.maximum(
                jnp.dot(pooled, wc1_ref[...], preferred_element_type=jnp.float32)
                + bc1_ref[...], 0.0)
            o_ref[...] = (jnp.dot(l1, wc2_ref[...],
                                  preferred_element_type=jnp.float32)
                          + bc2_ref[...])

    full = lambda shape: pl.BlockSpec(shape, lambda i: tuple(0 for _ in shape))
    return pl.pallas_call(
        body,
        grid=(NBLK,),
        in_specs=[
            pl.BlockSpec((BLK, HIDDEN), lambda i: (i, 0)),
            full((HIDDEN, 128)),
            full((1, 128)),
            full((128, 128)),
            full((1, 128)),
        ],
        out_specs=pl.BlockSpec((1, 128), lambda i: (0, 0)),
        out_shape=jax.ShapeDtypeStruct((1, 128), jnp.float32),
        scratch_shapes=[pltpu.VMEM((1, HIDDEN), jnp.float32)],
    )(h, wc1, bc1, wc2, bc2)


def _fold_bn(p):
    s = p['gamma'] * jax.lax.rsqrt(p['rv'] + 1e-5)
    w2 = p['W2'] * s[None, :]
    b2 = p['b2'] * s + p['beta'] - p['rm'] * s
    return w2, b2


def kernel(x, edge_attr, params, edge_index, batch):
    row = edge_index[0]
    col = edge_index[1]
    layers = params['layers']

    pe, pc = _edge_agg_sc(row, edge_attr)

    p0 = layers[0]
    u1 = _u1_tc(x, pe, pc, p0['W1'][:D_FEAT], p0['W1'][D_FEAT:])

    w2b2 = [_fold_bn(p) for p in layers]
    b1s = jnp.stack([p['b1'][None, :] for p in layers])
    w2s = jnp.stack([w for w, _ in w2b2])
    b2s = jnp.stack([b[None, :] for _, b in w2b2])
    w1n = jnp.stack([layers[1]['W1'], layers[2]['W1'],
                     jnp.eye(HIDDEN, dtype=jnp.float32)])

    def step(u, ws):
        b1_i, w2_i, b2_i, w1n_i = ws
        s = _seg_sum_sc(row, col, u)
        un = _layer_tc(u, s, b1_i, w2_i, b2_i, w1n_i)
        return un, None

    h3, _ = lax.scan(step, u1, (b1s, w2s, b2s, w1n))

    c = params['clf']
    wc1 = jnp.zeros((HIDDEN, 128), jnp.float32).at[:, :HIDDEN // 2].set(c['W1'])
    bc1 = jnp.zeros((1, 128), jnp.float32).at[0, :HIDDEN // 2].set(c['b1'])
    wc2 = jnp.zeros((128, 128), jnp.float32).at[:HIDDEN // 2, :2].set(c['W2'])
    bc2 = jnp.zeros((1, 128), jnp.float32).at[0, :2].set(c['b2'])
    out = _pool_clf_tc(h3, wc1, bc1, wc2, bc2)
    return out[:, :2]
